# baseline (speedup 1.0000x reference)
import functools

import jax
import jax.numpy as jnp
from jax import lax
from jax.experimental import pallas as pl
from jax.experimental.pallas import tpu as pltpu
from jax.experimental.pallas import tpu_sc as plsc

NUM_CATS = 16
EPS = 0.1
B, N, H, W = 4, 64, 256, 256
HCHUNK = 32
NCHUNKS = H // HCHUNK
L = 16


def _reduce_body(p_ref, g_ref, w_ref, inter_ref, union_ref, gmax_ref):
    c = pl.program_id(1)
    p = p_ref[0]
    g = g_ref[0]
    w = w_ref[0]
    pg = p * g
    s = p + g
    inter = jnp.sum(pg * w, axis=(1, 2))
    union = jnp.sum(s * w, axis=(1, 2)) - inter
    gm = jnp.max(g, axis=(1, 2))

    @pl.when(c == 0)
    def _init():
        inter_ref[0, 0, :] = inter
        union_ref[0, 0, :] = union
        gmax_ref[0, 0, :] = gm

    @pl.when(c != 0)
    def _acc():
        inter_ref[0, 0, :] += inter
        union_ref[0, 0, :] += union
        gmax_ref[0, 0, :] = jnp.maximum(gmax_ref[0, 0, :], gm)


def _spatial_reduce(pred, gt, weights):
    weights = weights.reshape(B, 1, H, W)
    out_sds = jax.ShapeDtypeStruct((B, 1, N), jnp.float32)
    io_spec = pl.BlockSpec((1, N, HCHUNK, W), lambda b, c: (b, 0, c, 0))
    w_spec = pl.BlockSpec((1, 1, HCHUNK, W), lambda b, c: (b, 0, c, 0))
    o_spec = pl.BlockSpec((1, 1, N), lambda b, c: (b, 0, 0))
    return pl.pallas_call(
        _reduce_body,
        grid=(B, NCHUNKS),
        in_specs=[io_spec, io_spec, w_spec],
        out_specs=[o_spec, o_spec, o_spec],
        out_shape=[out_sds, out_sds, out_sds],
    )(pred, gt, weights)


def _lane_perm(x, perm):
    return lax.gather(
        x, perm[:, None],
        lax.GatherDimensionNumbers(offset_dims=(), collapsed_slice_dims=(0,),
                                   start_index_map=(0,)),
        (1,), mode=lax.GatherScatterMode.PROMISE_IN_BOUNDS)


def _vsum(x, li):
    for shift in (8, 4, 2, 1):
        x = x + _lane_perm(x, jnp.bitwise_and(li + shift, L - 1))
    return x


def _epilogue_body(inter_h, union_h, gmax_h, fg_h, cat_h, o0_h, o1_h,
                   inter_v, union_v, gmax_v, fg_v, cat_v, o0_v, o1_v):
    cid = lax.axis_index("c")
    sid = lax.axis_index("s")

    @pl.when((cid == 0) & (sid == 0))
    def _():
        pltpu.sync_copy(inter_h, inter_v)
        pltpu.sync_copy(union_h, union_v)
        pltpu.sync_copy(gmax_h, gmax_v)
        pltpu.sync_copy(fg_h, fg_v)
        pltpu.sync_copy(cat_h, cat_v)

        part_num = [jnp.zeros((L,), jnp.float32) for _ in range(NUM_CATS)]
        part_den = [jnp.zeros((L,), jnp.float32) for _ in range(NUM_CATS)]
        zero = jnp.zeros((L,), jnp.float32)
        for i in range(B):
            for j in range(N // L):
                sl = pl.ds(j * L, L)
                it = inter_v[i, 0, sl]
                un = union_v[i, 0, sl]
                gm = gmax_v[i, 0, sl]
                fg = fg_v[i, sl]
                kt = cat_v[i, sl]
                iou = it / (un + 1e-6)
                x2 = iou * iou
                x4 = x2 * x2
                y = 1.0 - iou
                y2 = y * y
                y4 = y2 * y2
                tp = x4 / (x4 + y4)
                fp = 1.0 - tp
                gnz = jnp.where(gm > 0.0, 1.0, 0.0)
                tpi = tp * fg * gnz
                numt = tpi * iou
                dent = tpi + 0.5 * (fp * gnz) + 0.5 * ((1.0 - gnz) * fp * fg)
                for c in range(NUM_CATS):
                    m = kt == c
                    part_num[c] = part_num[c] + jnp.where(m, numt, zero)
                    part_den[c] = part_den[c] + jnp.where(m, dent, zero)

        li = lax.iota(jnp.int32, L)
        num_acc = jnp.zeros((L,), jnp.float32)
        den_acc = jnp.zeros((L,), jnp.float32)
        for c in range(NUM_CATS):
            num_acc = num_acc + jnp.where(li == c, _vsum(part_num[c], li), zero)
            den_acc = den_acc + jnp.where(li == c, _vsum(part_den[c], li), zero)

        validf = jnp.where(den_acc > 0.0, 1.0, 0.0)
        pq = (num_acc + EPS) / (den_acc + EPS)
        pqv = validf * pq
        n_valid = _vsum(validf, li)
        full_pq = _vsum(pqv, li) / n_valid
        o0_v[...] = 1.0 - full_pq
        o1_v[...] = pqv / (full_pq * n_valid + 1e-06)
        pltpu.sync_copy(o0_v, o0_h)
        pltpu.sync_copy(o1_v, o1_h)


@functools.lru_cache(maxsize=1)
def _build_epilogue():
    return functools.partial(
        pl.kernel,
        out_type=[jax.ShapeDtypeStruct((L,), jnp.float32),
                  jax.ShapeDtypeStruct((NUM_CATS,), jnp.float32)],
        mesh=plsc.VectorSubcoreMesh(core_axis_name="c", subcore_axis_name="s"),
        scratch_types=[pltpu.VMEM((B, 1, N), jnp.float32),
                       pltpu.VMEM((B, 1, N), jnp.float32),
                       pltpu.VMEM((B, 1, N), jnp.float32),
                       pltpu.VMEM((B, N), jnp.float32),
                       pltpu.VMEM((B, N), jnp.int32),
                       pltpu.VMEM((L,), jnp.float32),
                       pltpu.VMEM((L,), jnp.float32)],
    )(_epilogue_body)


def kernel(pan_pred_batch, pan_gt_batch, weights, foreground_prob, category_ids):
    inter, union, gmax = _spatial_reduce(pan_pred_batch, pan_gt_batch, weights)
    o0, o1 = _build_epilogue()(inter, union, gmax, foreground_prob, category_ids)
    return (o0[0], o1)


# speedup vs baseline: 1.1116x; 1.1116x over previous
import functools

import jax
import jax.numpy as jnp
from jax import lax
from jax.experimental import pallas as pl
from jax.experimental.pallas import tpu as pltpu
from jax.experimental.pallas import tpu_sc as plsc

NUM_CATS = 16
EPS = 0.1
B, N, H, W = 4, 64, 256, 256
HCHUNK = 128
NCHUNKS = H // HCHUNK
L = 16


def _reduce_body(p_ref, g_ref, w_ref, inter_ref, union_ref, gmax_ref):
    c = pl.program_id(1)
    p = p_ref[0]
    g = g_ref[0]
    w = w_ref[0]
    pg = p * g
    s = p + g
    inter = jnp.sum(pg * w, axis=(1, 2))
    union = jnp.sum(s * w, axis=(1, 2)) - inter
    gm = jnp.max(g, axis=(1, 2))

    @pl.when(c == 0)
    def _init():
        inter_ref[0, 0, :] = inter
        union_ref[0, 0, :] = union
        gmax_ref[0, 0, :] = gm

    @pl.when(c != 0)
    def _acc():
        inter_ref[0, 0, :] += inter
        union_ref[0, 0, :] += union
        gmax_ref[0, 0, :] = jnp.maximum(gmax_ref[0, 0, :], gm)


def _spatial_reduce(pred, gt, weights):
    weights = weights.reshape(B, 1, H, W)
    out_sds = jax.ShapeDtypeStruct((B, 1, N), jnp.float32)
    io_spec = pl.BlockSpec((1, N, HCHUNK, W), lambda b, c: (b, 0, c, 0))
    w_spec = pl.BlockSpec((1, 1, HCHUNK, W), lambda b, c: (b, 0, c, 0))
    o_spec = pl.BlockSpec((1, 1, N), lambda b, c: (b, 0, 0))
    return pl.pallas_call(
        _reduce_body,
        grid=(B, NCHUNKS),
        in_specs=[io_spec, io_spec, w_spec],
        out_specs=[o_spec, o_spec, o_spec],
        out_shape=[out_sds, out_sds, out_sds],
    )(pred, gt, weights)


def _lane_perm(x, perm):
    return lax.gather(
        x, perm[:, None],
        lax.GatherDimensionNumbers(offset_dims=(), collapsed_slice_dims=(0,),
                                   start_index_map=(0,)),
        (1,), mode=lax.GatherScatterMode.PROMISE_IN_BOUNDS)


def _vsum(x, li):
    for shift in (8, 4, 2, 1):
        x = x + _lane_perm(x, jnp.bitwise_and(li + shift, L - 1))
    return x


def _epilogue_body(inter_h, union_h, gmax_h, fg_h, cat_h, o0_h, o1_h,
                   inter_v, union_v, gmax_v, fg_v, cat_v, o0_v, o1_v):
    cid = lax.axis_index("c")
    sid = lax.axis_index("s")

    @pl.when((cid == 0) & (sid == 0))
    def _():
        pltpu.sync_copy(inter_h, inter_v)
        pltpu.sync_copy(union_h, union_v)
        pltpu.sync_copy(gmax_h, gmax_v)
        pltpu.sync_copy(fg_h, fg_v)
        pltpu.sync_copy(cat_h, cat_v)

        part_num = [jnp.zeros((L,), jnp.float32) for _ in range(NUM_CATS)]
        part_den = [jnp.zeros((L,), jnp.float32) for _ in range(NUM_CATS)]
        zero = jnp.zeros((L,), jnp.float32)
        for i in range(B):
            for j in range(N // L):
                sl = pl.ds(j * L, L)
                it = inter_v[i, 0, sl]
                un = union_v[i, 0, sl]
                gm = gmax_v[i, 0, sl]
                fg = fg_v[i, sl]
                kt = cat_v[i, sl]
                iou = it / (un + 1e-6)
                x2 = iou * iou
                x4 = x2 * x2
                y = 1.0 - iou
                y2 = y * y
                y4 = y2 * y2
                tp = x4 / (x4 + y4)
                fp = 1.0 - tp
                gnz = jnp.where(gm > 0.0, 1.0, 0.0)
                tpi = tp * fg * gnz
                numt = tpi * iou
                dent = tpi + 0.5 * (fp * gnz) + 0.5 * ((1.0 - gnz) * fp * fg)
                for c in range(NUM_CATS):
                    m = kt == c
                    part_num[c] = part_num[c] + jnp.where(m, numt, zero)
                    part_den[c] = part_den[c] + jnp.where(m, dent, zero)

        li = lax.iota(jnp.int32, L)
        num_acc = jnp.zeros((L,), jnp.float32)
        den_acc = jnp.zeros((L,), jnp.float32)
        for c in range(NUM_CATS):
            num_acc = num_acc + jnp.where(li == c, _vsum(part_num[c], li), zero)
            den_acc = den_acc + jnp.where(li == c, _vsum(part_den[c], li), zero)

        validf = jnp.where(den_acc > 0.0, 1.0, 0.0)
        pq = (num_acc + EPS) / (den_acc + EPS)
        pqv = validf * pq
        n_valid = _vsum(validf, li)
        full_pq = _vsum(pqv, li) / n_valid
        o0_v[...] = 1.0 - full_pq
        o1_v[...] = pqv / (full_pq * n_valid + 1e-06)
        pltpu.sync_copy(o0_v, o0_h)
        pltpu.sync_copy(o1_v, o1_h)


@functools.lru_cache(maxsize=1)
def _build_epilogue():
    return functools.partial(
        pl.kernel,
        out_type=[jax.ShapeDtypeStruct((L,), jnp.float32),
                  jax.ShapeDtypeStruct((NUM_CATS,), jnp.float32)],
        mesh=plsc.VectorSubcoreMesh(core_axis_name="c", subcore_axis_name="s"),
        scratch_types=[pltpu.VMEM((B, 1, N), jnp.float32),
                       pltpu.VMEM((B, 1, N), jnp.float32),
                       pltpu.VMEM((B, 1, N), jnp.float32),
                       pltpu.VMEM((B, N), jnp.float32),
                       pltpu.VMEM((B, N), jnp.int32),
                       pltpu.VMEM((L,), jnp.float32),
                       pltpu.VMEM((L,), jnp.float32)],
    )(_epilogue_body)


def kernel(pan_pred_batch, pan_gt_batch, weights, foreground_prob, category_ids):
    inter, union, gmax = _spatial_reduce(pan_pred_batch, pan_gt_batch, weights)
    o0, o1 = _build_epilogue()(inter, union, gmax, foreground_prob, category_ids)
    return (o0[0], o1)


# speedup vs baseline: 1.1860x; 1.0669x over previous
import functools

import jax
import jax.numpy as jnp
from jax import lax
from jax.experimental import pallas as pl
from jax.experimental.pallas import tpu as pltpu
from jax.experimental.pallas import tpu_sc as plsc

NUM_CATS = 16
EPS = 0.1
B, N, H, W = 4, 64, 256, 256
HCHUNK = 64
NCHUNKS = H // HCHUNK
L = 16


def _reduce_body(p_ref, g_ref, w_ref, inter_ref, union_ref, gmax_ref):
    c = pl.program_id(1)
    p = p_ref[0]
    g = g_ref[0]
    w = w_ref[0]
    pg = p * g
    s = p + g
    inter = jnp.sum(pg * w, axis=(1, 2))
    union = jnp.sum(s * w, axis=(1, 2)) - inter
    gm = jnp.max(g, axis=(1, 2))

    @pl.when(c == 0)
    def _init():
        inter_ref[0, 0, :] = inter
        union_ref[0, 0, :] = union
        gmax_ref[0, 0, :] = gm

    @pl.when(c != 0)
    def _acc():
        inter_ref[0, 0, :] += inter
        union_ref[0, 0, :] += union
        gmax_ref[0, 0, :] = jnp.maximum(gmax_ref[0, 0, :], gm)


def _spatial_reduce(pred, gt, weights):
    weights = weights.reshape(B, 1, H, W)
    out_sds = jax.ShapeDtypeStruct((B, 1, N), jnp.float32)
    io_spec = pl.BlockSpec((1, N, HCHUNK, W), lambda b, c: (b, 0, c, 0))
    w_spec = pl.BlockSpec((1, 1, HCHUNK, W), lambda b, c: (b, 0, c, 0))
    o_spec = pl.BlockSpec((1, 1, N), lambda b, c: (b, 0, 0))
    return pl.pallas_call(
        _reduce_body,
        grid=(B, NCHUNKS),
        in_specs=[io_spec, io_spec, w_spec],
        out_specs=[o_spec, o_spec, o_spec],
        out_shape=[out_sds, out_sds, out_sds],
    )(pred, gt, weights)


def _lane_perm(x, perm):
    return lax.gather(
        x, perm[:, None],
        lax.GatherDimensionNumbers(offset_dims=(), collapsed_slice_dims=(0,),
                                   start_index_map=(0,)),
        (1,), mode=lax.GatherScatterMode.PROMISE_IN_BOUNDS)


def _vsum(x, li):
    for shift in (8, 4, 2, 1):
        x = x + _lane_perm(x, jnp.bitwise_and(li + shift, L - 1))
    return x


def _epilogue_body(inter_h, union_h, gmax_h, fg_h, cat_h, o0_h, o1_h,
                   inter_v, union_v, gmax_v, fg_v, cat_v, o0_v, o1_v, sem):
    cid = lax.axis_index("c")
    sid = lax.axis_index("s")

    @pl.when((cid == 0) & (sid == 0))
    def _():
        c1 = pltpu.async_copy(inter_h, inter_v, sem)
        c2 = pltpu.async_copy(union_h, union_v, sem)
        c3 = pltpu.async_copy(gmax_h, gmax_v, sem)
        c4 = pltpu.async_copy(fg_h, fg_v, sem)
        c5 = pltpu.async_copy(cat_h, cat_v, sem)
        c1.wait()
        c2.wait()
        c3.wait()
        c4.wait()
        c5.wait()

        part_num = [jnp.zeros((L,), jnp.float32) for _ in range(NUM_CATS)]
        part_den = [jnp.zeros((L,), jnp.float32) for _ in range(NUM_CATS)]
        zero = jnp.zeros((L,), jnp.float32)
        for i in range(B):
            for j in range(N // L):
                sl = pl.ds(j * L, L)
                it = inter_v[i, 0, sl]
                un = union_v[i, 0, sl]
                gm = gmax_v[i, 0, sl]
                fg = fg_v[i, sl]
                kt = cat_v[i, sl]
                iou = it / (un + 1e-6)
                x2 = iou * iou
                x4 = x2 * x2
                y = 1.0 - iou
                y2 = y * y
                y4 = y2 * y2
                tp = x4 / (x4 + y4)
                fp = 1.0 - tp
                gnz = jnp.where(gm > 0.0, 1.0, 0.0)
                tpi = tp * fg * gnz
                numt = tpi * iou
                dent = tpi + 0.5 * (fp * gnz) + 0.5 * ((1.0 - gnz) * fp * fg)
                for c in range(NUM_CATS):
                    m = kt == c
                    part_num[c] = part_num[c] + jnp.where(m, numt, zero)
                    part_den[c] = part_den[c] + jnp.where(m, dent, zero)

        li = lax.iota(jnp.int32, L)
        num_acc = jnp.zeros((L,), jnp.float32)
        den_acc = jnp.zeros((L,), jnp.float32)
        for c in range(NUM_CATS):
            num_acc = num_acc + jnp.where(li == c, _vsum(part_num[c], li), zero)
            den_acc = den_acc + jnp.where(li == c, _vsum(part_den[c], li), zero)

        validf = jnp.where(den_acc > 0.0, 1.0, 0.0)
        pq = (num_acc + EPS) / (den_acc + EPS)
        pqv = validf * pq
        n_valid = _vsum(validf, li)
        full_pq = _vsum(pqv, li) / n_valid
        o0_v[...] = 1.0 - full_pq
        o1_v[...] = pqv / (full_pq * n_valid + 1e-06)
        pltpu.sync_copy(o0_v, o0_h)
        pltpu.sync_copy(o1_v, o1_h)


@functools.lru_cache(maxsize=1)
def _build_epilogue():
    return functools.partial(
        pl.kernel,
        out_type=[jax.ShapeDtypeStruct((L,), jnp.float32),
                  jax.ShapeDtypeStruct((NUM_CATS,), jnp.float32)],
        mesh=plsc.VectorSubcoreMesh(core_axis_name="c", subcore_axis_name="s"),
        scratch_types=[pltpu.VMEM((B, 1, N), jnp.float32),
                       pltpu.VMEM((B, 1, N), jnp.float32),
                       pltpu.VMEM((B, 1, N), jnp.float32),
                       pltpu.VMEM((B, N), jnp.float32),
                       pltpu.VMEM((B, N), jnp.int32),
                       pltpu.VMEM((L,), jnp.float32),
                       pltpu.VMEM((L,), jnp.float32),
                       pltpu.SemaphoreType.DMA],
    )(_epilogue_body)


def kernel(pan_pred_batch, pan_gt_batch, weights, foreground_prob, category_ids):
    inter, union, gmax = _spatial_reduce(pan_pred_batch, pan_gt_batch, weights)
    o0, o1 = _build_epilogue()(inter, union, gmax, foreground_prob, category_ids)
    return (o0[0], o1)


# speedup vs baseline: 1.1922x; 1.0053x over previous
import functools

import jax
import jax.numpy as jnp
from jax import lax
from jax.experimental import pallas as pl
from jax.experimental.pallas import tpu as pltpu
from jax.experimental.pallas import tpu_sc as plsc

NUM_CATS = 16
EPS = 0.1
B, N, H, W = 4, 64, 256, 256
HCHUNK = 64
NCHUNKS = H // HCHUNK
L = 16


def _reduce_body(p_ref, g_ref, w_ref, inter_ref, union_ref, gmax_ref):
    c = pl.program_id(1)
    p = p_ref[0]
    g = g_ref[0]
    w = w_ref[0]
    pg = p * g
    s = p + g
    inter = jnp.sum(pg * w, axis=(1, 2))
    union = jnp.sum(s * w, axis=(1, 2)) - inter
    gm = jnp.max(g, axis=(1, 2))

    @pl.when(c == 0)
    def _init():
        inter_ref[0, 0, :] = inter
        union_ref[0, 0, :] = union
        gmax_ref[0, 0, :] = gm

    @pl.when(c != 0)
    def _acc():
        inter_ref[0, 0, :] += inter
        union_ref[0, 0, :] += union
        gmax_ref[0, 0, :] = jnp.maximum(gmax_ref[0, 0, :], gm)


def _spatial_reduce(pred, gt, weights):
    weights = weights.reshape(B, 1, H, W)
    out_sds = jax.ShapeDtypeStruct((B, 1, N), jnp.float32)
    io_spec = pl.BlockSpec((1, N, HCHUNK, W), lambda b, c: (b, 0, c, 0))
    w_spec = pl.BlockSpec((1, 1, HCHUNK, W), lambda b, c: (b, 0, c, 0))
    o_spec = pl.BlockSpec((1, 1, N), lambda b, c: (b, 0, 0))
    return pl.pallas_call(
        _reduce_body,
        grid=(B, NCHUNKS),
        in_specs=[io_spec, io_spec, w_spec],
        out_specs=[o_spec, o_spec, o_spec],
        out_shape=[out_sds, out_sds, out_sds],
    )(pred, gt, weights)


def _lane_perm(x, perm):
    return lax.gather(
        x, perm[:, None],
        lax.GatherDimensionNumbers(offset_dims=(), collapsed_slice_dims=(0,),
                                   start_index_map=(0,)),
        (1,), mode=lax.GatherScatterMode.PROMISE_IN_BOUNDS)


def _vsum(x, li):
    for shift in (8, 4, 2, 1):
        x = x + _lane_perm(x, jnp.bitwise_and(li + shift, L - 1))
    return x


def _epilogue_body(inter_h, union_h, gmax_h, fg_h, cat_h, o0_h, o1_h,
                   inter_v, union_v, gmax_v, fg_v, cat_v, o0_v, o1_v, sem):
    cid = lax.axis_index("c")
    sid = lax.axis_index("s")

    @pl.when((cid == 0) & (sid == 0))
    def _():
        c1 = pltpu.async_copy(inter_h, inter_v, sem)
        c2 = pltpu.async_copy(union_h, union_v, sem)
        c3 = pltpu.async_copy(gmax_h, gmax_v, sem)
        c4 = pltpu.async_copy(fg_h, fg_v, sem)
        c5 = pltpu.async_copy(cat_h, cat_v, sem)
        c1.wait()
        c2.wait()
        c3.wait()
        c4.wait()
        c5.wait()

        part_num = [jnp.zeros((L,), jnp.float32) for _ in range(NUM_CATS)]
        part_den = [jnp.zeros((L,), jnp.float32) for _ in range(NUM_CATS)]
        zero = jnp.zeros((L,), jnp.float32)
        for i in range(B):
            for j in range(N // L):
                sl = pl.ds(j * L, L)
                it = inter_v[i, 0, sl]
                un = union_v[i, 0, sl]
                gm = gmax_v[i, 0, sl]
                fg = fg_v[i, sl]
                kt = cat_v[i, sl]
                iou = it / (un + 1e-6)
                x2 = iou * iou
                x4 = x2 * x2
                y = 1.0 - iou
                y2 = y * y
                y4 = y2 * y2
                tp = x4 / (x4 + y4)
                fp = 1.0 - tp
                gnz = jnp.where(gm > 0.0, 1.0, 0.0)
                tpi = tp * fg * gnz
                numt = tpi * iou
                dent = tpi + 0.5 * (fp * gnz) + 0.5 * ((1.0 - gnz) * fp * fg)
                for c in range(NUM_CATS):
                    m = kt == c
                    part_num[c] = part_num[c] + jnp.where(m, numt, zero)
                    part_den[c] = part_den[c] + jnp.where(m, dent, zero)

        li = lax.iota(jnp.int32, L)
        num_acc = jnp.zeros((L,), jnp.float32)
        den_acc = jnp.zeros((L,), jnp.float32)
        for c in range(NUM_CATS):
            num_acc = num_acc + jnp.where(li == c, _vsum(part_num[c], li), zero)
            den_acc = den_acc + jnp.where(li == c, _vsum(part_den[c], li), zero)

        validf = jnp.where(den_acc > 0.0, 1.0, 0.0)
        pq = (num_acc + EPS) / (den_acc + EPS)
        pqv = validf * pq
        n_valid = _vsum(validf, li)
        full_pq = _vsum(pqv, li) / n_valid
        o0_v[...] = 1.0 - full_pq
        o1_v[...] = pqv / (full_pq * n_valid + 1e-06)
        d1 = pltpu.async_copy(o0_v, o0_h, sem)
        d2 = pltpu.async_copy(o1_v, o1_h, sem)
        d1.wait()
        d2.wait()


@functools.lru_cache(maxsize=1)
def _build_epilogue():
    return functools.partial(
        pl.kernel,
        out_type=[jax.ShapeDtypeStruct((L,), jnp.float32),
                  jax.ShapeDtypeStruct((NUM_CATS,), jnp.float32)],
        mesh=plsc.VectorSubcoreMesh(core_axis_name="c", subcore_axis_name="s"),
        scratch_types=[pltpu.VMEM((B, 1, N), jnp.float32),
                       pltpu.VMEM((B, 1, N), jnp.float32),
                       pltpu.VMEM((B, 1, N), jnp.float32),
                       pltpu.VMEM((B, N), jnp.float32),
                       pltpu.VMEM((B, N), jnp.int32),
                       pltpu.VMEM((L,), jnp.float32),
                       pltpu.VMEM((L,), jnp.float32),
                       pltpu.SemaphoreType.DMA],
    )(_epilogue_body)


def kernel(pan_pred_batch, pan_gt_batch, weights, foreground_prob, category_ids):
    inter, union, gmax = _spatial_reduce(pan_pred_batch, pan_gt_batch, weights)
    o0, o1 = _build_epilogue()(inter, union, gmax, foreground_prob, category_ids)
    return (o0[0], o1)
